# baseline (speedup 1.0000x reference)
import jax
import jax.numpy as jnp
from jax.experimental import pallas as pl
from jax.experimental.pallas import tpu as pltpu

_SCORE_THR = 0.05
_TOPK = 1000
_NMS_THR = 0.5
_DET = 300


def _decode_body(feats_ref, score_ref, label_ref, bbox_ref):
    x = feats_ref[0]
    cls = jax.nn.sigmoid(x[:, 4:])
    mx = jnp.max(cls, axis=-1)
    iota = jax.lax.broadcasted_iota(jnp.int32, cls.shape, 1)
    lab = jnp.min(jnp.where(cls == mx[:, None], iota, 10000), axis=-1)
    masked = jnp.where(mx > _SCORE_THR, mx, -1.0)
    score_ref[0, 0] = masked
    label_ref[0, 0] = lab.astype(jnp.float32)
    bbox_ref[0] = jax.nn.sigmoid(x[:, :4])


def _nms_body(scale_ref, boxes_ref, labels_ref, scores_ref, out_ref, m_ref):
    B = boxes_ref.shape[0]
    K = _TOPK
    jj = jax.lax.broadcasted_iota(jnp.int32, (K, K), 0)
    ii = jax.lax.broadcasted_iota(jnp.int32, (K, K), 1)

    def clipped_boxes(b):
        w = scale_ref[b, 0]
        h = scale_ref[b, 1]
        tb = boxes_ref[b]
        x1 = jnp.clip(tb[:, 0] * w, 0.0, w)
        y1 = jnp.clip(tb[:, 1] * h, 0.0, h)
        x2 = jnp.clip(tb[:, 2] * w, 0.0, w)
        y2 = jnp.clip(tb[:, 3] * h, 0.0, h)
        return x1, y1, x2, y2

    for b in range(B):
        x1, y1, x2, y2 = clipped_boxes(b)
        max_coord = jnp.max(jnp.stack([x1, y1, x2, y2], axis=1)) + 1.0
        off = labels_ref[b] * max_coord
        ox1 = x1 + off
        oy1 = y1 + off
        ox2 = x2 + off
        oy2 = y2 + off
        area = jnp.clip(ox2 - ox1, 0.0, None) * jnp.clip(oy2 - oy1, 0.0, None)
        xx1 = jnp.maximum(ox1[:, None], ox1[None, :])
        yy1 = jnp.maximum(oy1[:, None], oy1[None, :])
        xx2 = jnp.minimum(ox2[:, None], ox2[None, :])
        yy2 = jnp.minimum(oy2[:, None], oy2[None, :])
        inter = jnp.clip(xx2 - xx1, 0.0, None) * jnp.clip(yy2 - yy1, 0.0, None)
        union = area[:, None] + area[None, :] - inter
        iou = inter / jnp.maximum(union, 1e-9)
        m = jnp.where(jnp.logical_and(iou > _NMS_THR, jj < ii), 1.0, 0.0)
        m_ref[b] = m.astype(jnp.bfloat16)

    valid = (scores_ref[...] > _SCORE_THR).astype(jnp.float32)

    def cond(c):
        it, _, changed = c
        return jnp.logical_and(changed, it < K)

    def body(c):
        it, k, _ = c
        rows = []
        for b in range(B):
            kb = k[b : b + 1].astype(jnp.bfloat16)
            rows.append(
                jax.lax.dot_general(
                    kb,
                    m_ref[b],
                    (((1,), (0,)), ((), ())),
                    preferred_element_type=jnp.float32,
                )
            )
        s = jnp.concatenate(rows, axis=0)
        knew = jnp.where(jnp.logical_and(valid > 0.0, s == 0.0), 1.0, 0.0)
        changed = jnp.any(knew != k)
        return (it + 1, knew, changed)

    _, keep, _ = jax.lax.while_loop(
        cond, body, (jnp.int32(0), valid, jnp.bool_(True))
    )

    upper = jnp.where(jj <= ii, 1.0, 0.0).astype(jnp.bfloat16)
    rr = jax.lax.broadcasted_iota(jnp.int32, (_DET, K), 0).astype(jnp.float32)
    for b in range(B):
        kb = keep[b : b + 1]
        cum = jax.lax.dot_general(
            kb.astype(jnp.bfloat16),
            upper,
            (((1,), (0,)), ((), ())),
            preferred_element_type=jnp.float32,
        )
        dest = cum - 1.0
        p = jnp.where(
            jnp.logical_and(kb > 0.0, dest == rr), 1.0, 0.0
        )
        x1, y1, x2, y2 = clipped_boxes(b)
        vals = jnp.stack(
            [x1, y1, x2, y2, scores_ref[b], labels_ref[b]], axis=1
        )
        out_ref[b] = jax.lax.dot_general(
            p, vals, (((1,), (0,)), ((), ())), preferred_element_type=jnp.float32
        )


def kernel(feats, image_shapes):
    B, N, C = feats.shape
    R = 2000
    nchunk = N // R

    masked3, label3, bbox = pl.pallas_call(
        _decode_body,
        grid=(B, nchunk),
        in_specs=[pl.BlockSpec((1, R, C), lambda b, c: (b, c, 0))],
        out_specs=[
            pl.BlockSpec((1, 1, R), lambda b, c: (b * nchunk + c, 0, 0)),
            pl.BlockSpec((1, 1, R), lambda b, c: (b * nchunk + c, 0, 0)),
            pl.BlockSpec((1, R, 4), lambda b, c: (b, c, 0)),
        ],
        out_shape=[
            jax.ShapeDtypeStruct((B * nchunk, 1, R), jnp.float32),
            jax.ShapeDtypeStruct((B * nchunk, 1, R), jnp.float32),
            jax.ShapeDtypeStruct((B, N, 4), jnp.float32),
        ],
    )(feats)
    masked = masked3.reshape(B, N)
    label = label3.reshape(B, N)

    top_scores, top_idx = jax.lax.top_k(masked, _TOPK)
    tb = jnp.take_along_axis(bbox, top_idx[..., None], axis=1)
    tl = jnp.take_along_axis(label, top_idx, axis=1)

    wh = image_shapes.astype(jnp.float32)
    scales = jnp.stack([wh[:, 1], wh[:, 0]], axis=1)

    return pl.pallas_call(
        _nms_body,
        grid=(1,),
        in_specs=[
            pl.BlockSpec((B, 2), lambda g: (0, 0)),
            pl.BlockSpec((B, _TOPK, 4), lambda g: (0, 0, 0)),
            pl.BlockSpec((B, _TOPK), lambda g: (0, 0)),
            pl.BlockSpec((B, _TOPK), lambda g: (0, 0)),
        ],
        out_specs=pl.BlockSpec((B, _DET, 6), lambda g: (0, 0, 0)),
        out_shape=jax.ShapeDtypeStruct((B, _DET, 6), jnp.float32),
        scratch_shapes=[pltpu.VMEM((B, _TOPK, _TOPK), jnp.bfloat16)],
    )(scales, tb, tl, top_scores)


# speedup vs baseline: 1.0099x; 1.0099x over previous
import jax
import jax.numpy as jnp
from jax.experimental import pallas as pl

_SCORE_THR = 0.05
_TOPK = 1000
_NMS_THR = 0.5
_DET = 300


def _decode_body(feats_ref, score_ref, label_ref, bbox_ref):
    x = feats_ref[0]
    cls = jax.nn.sigmoid(x[:, 4:])
    mx = jnp.max(cls, axis=-1)
    iota = jax.lax.broadcasted_iota(jnp.int32, cls.shape, 1)
    lab = jnp.min(jnp.where(cls == mx[:, None], iota, 10000), axis=-1)
    masked = jnp.where(mx > _SCORE_THR, mx, -1.0)
    score_ref[0, 0] = masked
    label_ref[0, 0] = lab.astype(jnp.float32)
    bbox_ref[0] = jax.nn.sigmoid(x[:, :4])


def _nms_body(scale_ref, boxes_ref, labels_ref, scores_ref, out_ref):
    w = scale_ref[0, 0, 0]
    h = scale_ref[0, 0, 1]
    tb = boxes_ref[0]
    lab = labels_ref[0, 0]
    sc = scores_ref[0, 0]
    K = tb.shape[0]

    x1 = jnp.clip(tb[:, 0] * w, 0.0, w)
    y1 = jnp.clip(tb[:, 1] * h, 0.0, h)
    x2 = jnp.clip(tb[:, 2] * w, 0.0, w)
    y2 = jnp.clip(tb[:, 3] * h, 0.0, h)
    bfull = jnp.stack([x1, y1, x2, y2], axis=1)

    max_coord = jnp.max(bfull) + 1.0
    off = lab * max_coord
    ox1 = x1 + off
    oy1 = y1 + off
    ox2 = x2 + off
    oy2 = y2 + off

    area = jnp.clip(ox2 - ox1, 0.0, None) * jnp.clip(oy2 - oy1, 0.0, None)
    xx1 = jnp.maximum(ox1[:, None], ox1[None, :])
    yy1 = jnp.maximum(oy1[:, None], oy1[None, :])
    xx2 = jnp.minimum(ox2[:, None], ox2[None, :])
    yy2 = jnp.minimum(oy2[:, None], oy2[None, :])
    inter = jnp.clip(xx2 - xx1, 0.0, None) * jnp.clip(yy2 - yy1, 0.0, None)
    union = area[:, None] + area[None, :] - inter
    iou = inter / jnp.maximum(union, 1e-9)

    jj = jax.lax.broadcasted_iota(jnp.int32, (K, K), 0)
    ii = jax.lax.broadcasted_iota(jnp.int32, (K, K), 1)
    m = jnp.where(jnp.logical_and(iou > _NMS_THR, jj < ii), 1.0, 0.0)

    valid = (sc > _SCORE_THR).astype(jnp.float32).reshape(1, K)

    def cond(c):
        it, _, changed = c
        return jnp.logical_and(changed, it < K)

    def body(c):
        it, k, _ = c
        s = jax.lax.dot_general(
            k, m, (((1,), (0,)), ((), ())), preferred_element_type=jnp.float32
        )
        knew = jnp.where(jnp.logical_and(valid > 0.0, s == 0.0), 1.0, 0.0)
        changed = jnp.any(knew != k)
        return (it + 1, knew, changed)

    _, keep, _ = jax.lax.while_loop(
        cond, body, (jnp.int32(0), valid, jnp.bool_(True))
    )

    upper = jnp.where(jj <= ii, 1.0, 0.0)
    cum = jax.lax.dot_general(
        keep, upper, (((1,), (0,)), ((), ())), preferred_element_type=jnp.float32
    )
    dest = cum - 1.0
    rr = jax.lax.broadcasted_iota(jnp.int32, (_DET, K), 0).astype(jnp.float32)
    p = jnp.where(
        jnp.logical_and(keep > 0.0, dest == rr), 1.0, 0.0
    )
    vals = jnp.concatenate(
        [bfull, sc[:, None], lab[:, None]], axis=1
    )
    out_ref[0] = jax.lax.dot_general(
        p, vals, (((1,), (0,)), ((), ())), preferred_element_type=jnp.float32
    )


def kernel(feats, image_shapes):
    B, N, C = feats.shape
    R = 2000
    nchunk = N // R

    masked3, label3, bbox = pl.pallas_call(
        _decode_body,
        grid=(B, nchunk),
        in_specs=[pl.BlockSpec((1, R, C), lambda b, c: (b, c, 0))],
        out_specs=[
            pl.BlockSpec((1, 1, R), lambda b, c: (b * nchunk + c, 0, 0)),
            pl.BlockSpec((1, 1, R), lambda b, c: (b * nchunk + c, 0, 0)),
            pl.BlockSpec((1, R, 4), lambda b, c: (b, c, 0)),
        ],
        out_shape=[
            jax.ShapeDtypeStruct((B * nchunk, 1, R), jnp.float32),
            jax.ShapeDtypeStruct((B * nchunk, 1, R), jnp.float32),
            jax.ShapeDtypeStruct((B, N, 4), jnp.float32),
        ],
    )(feats)
    masked = masked3.reshape(B, N)
    label = label3.reshape(B, N)

    top_scores, top_idx = jax.lax.top_k(masked, _TOPK)
    tb = jnp.take_along_axis(bbox, top_idx[..., None], axis=1)
    tl = jnp.take_along_axis(label, top_idx, axis=1)

    wh = image_shapes.astype(jnp.float32)
    scales = jnp.stack([wh[:, 1], wh[:, 0]], axis=1).reshape(B, 1, 2)

    return pl.pallas_call(
        _nms_body,
        grid=(B,),
        in_specs=[
            pl.BlockSpec((1, 1, 2), lambda b: (b, 0, 0)),
            pl.BlockSpec((1, _TOPK, 4), lambda b: (b, 0, 0)),
            pl.BlockSpec((1, 1, _TOPK), lambda b: (b, 0, 0)),
            pl.BlockSpec((1, 1, _TOPK), lambda b: (b, 0, 0)),
        ],
        out_specs=pl.BlockSpec((1, _DET, 6), lambda b: (b, 0, 0)),
        out_shape=jax.ShapeDtypeStruct((B, _DET, 6), jnp.float32),
    )(scales, tb, tl.reshape(B, 1, _TOPK), top_scores.reshape(B, 1, _TOPK))
